# baseline (speedup 1.0000x reference)
import functools

import jax
import jax.numpy as jnp
import numpy as np
from jax import lax
from jax.experimental import pallas as pl
from jax.experimental.pallas import tpu as pltpu
from jax.experimental.pallas import tpu_sc as plsc

NUM_STEPS = 3


def _msg_pre_body(h_ref, w1_ref, w2_ref, b_ref, a_ref, bb_ref):
    h = h_ref[...]
    a_ref[...] = jnp.dot(
        h, w1_ref[...], preferred_element_type=jnp.float32
    ).astype(jnp.bfloat16)
    bb_ref[...] = (
        jnp.dot(h, w2_ref[...], preferred_element_type=jnp.float32) + b_ref[...]
    ).astype(jnp.bfloat16)


def _node_body(h_ref, m_ref, w1_ref, w2_ref, b_ref, o_ref):
    m = m_ref[0] + m_ref[1]
    acc = jnp.dot(h_ref[...], w1_ref[...], preferred_element_type=jnp.float32)
    acc = acc + jnp.dot(m, w2_ref[...], preferred_element_type=jnp.float32)
    o_ref[...] = jnp.maximum(acc + b_ref[...], 0.0)


@functools.partial(jax.jit, static_argnames=("blk",))
def _msg_pre(h, w1, w2, b2d, blk):
    n, d = h.shape
    grid = (n // blk,)
    return pl.pallas_call(
        _msg_pre_body,
        grid=grid,
        in_specs=[
            pl.BlockSpec((blk, d), lambda i: (i, 0)),
            pl.BlockSpec((d, d), lambda i: (0, 0)),
            pl.BlockSpec((d, d), lambda i: (0, 0)),
            pl.BlockSpec((1, d), lambda i: (0, 0)),
        ],
        out_specs=[
            pl.BlockSpec((blk, d), lambda i: (i, 0)),
            pl.BlockSpec((blk, d), lambda i: (i, 0)),
        ],
        out_shape=[jax.ShapeDtypeStruct((n, d), jnp.bfloat16)] * 2,
    )(h, w1, w2, b2d)


@functools.partial(jax.jit, static_argnames=("blk",))
def _node_update(h, m2, w1, w2, b2d, blk):
    n, d = h.shape
    grid = (n // blk,)
    return pl.pallas_call(
        _node_body,
        grid=grid,
        in_specs=[
            pl.BlockSpec((blk, d), lambda i: (i, 0)),
            pl.BlockSpec((2, blk, d), lambda i: (0, i, 0)),
            pl.BlockSpec((d, d), lambda i: (0, 0)),
            pl.BlockSpec((d, d), lambda i: (0, 0)),
            pl.BlockSpec((1, d), lambda i: (0, 0)),
        ],
        out_specs=pl.BlockSpec((blk, d), lambda i: (i, 0)),
        out_shape=jax.ShapeDtypeStruct((n, d), jnp.float32),
    )(h, m2, w1, w2, b2d)


_NC = 2
_NS = 16
_LANES = 16
_ECHK = 80


def _make_sc_edge(n_nodes, n_edges, d):
    nw = _NC * _NS
    per_w = n_edges // nw
    n_chunks = per_w // _ECHK
    n_pairs = n_chunks // 2
    vecs_per_row = d // _LANES
    row_chk = _ECHK
    n_row_chunks = n_nodes // row_chk
    max_rc_per_sub = -(-n_row_chunks // _NS)

    mesh = plsc.VectorSubcoreMesh(core_axis_name="c", subcore_axis_name="s")

    @functools.partial(
        pl.kernel,
        mesh=mesh,
        compiler_params=pltpu.CompilerParams(
            needs_layout_passes=False, use_tc_tiling_on_sc=False),
        out_type=jax.ShapeDtypeStruct((_NC, n_nodes, d), jnp.float32),
        scratch_types=[
            pltpu.VMEM((2, _ECHK), jnp.int32),
            pltpu.VMEM((2, _ECHK), jnp.int32),
            pltpu.VMEM((_ECHK, d // 2), jnp.int32),
            pltpu.VMEM((_ECHK, d // 2), jnp.int32),
            pltpu.VMEM((_ECHK, d // 2), jnp.int32),
            pltpu.VMEM((_ECHK, d // 2), jnp.int32),
            pltpu.VMEM((_ECHK, d), jnp.float32),
            pltpu.VMEM((_ECHK, d), jnp.float32),
            pltpu.VMEM((2, _ECHK), jnp.int32),
            pltpu.VMEM_SHARED((n_nodes, d), jnp.float32),
            pltpu.SemaphoreType.DMA,
            pltpu.SemaphoreType.DMA,
            pltpu.SemaphoreType.DMA,
            pltpu.SemaphoreType.DMA,
            pltpu.SemaphoreType.DMA,
            pltpu.SemaphoreType.DMA,
            pltpu.SemaphoreType.DMA,
            pltpu.SemaphoreType.DMA,
        ],
    )
    def sc_edge(a_hbm, b_hbm, src_hbm, dst_hbm, out_hbm,
                idx_s2, idx_d2, buf_a0, buf_b0, buf_a1, buf_b1,
                buf_e0, buf_e1, sidx2, m_sh,
                sem_i0, sem_i1, sem_a0, sem_b0, sem_a1, sem_b1,
                sem_s0, sem_s1):
        c = lax.axis_index("c")
        s = lax.axis_index("s")
        wid = s * _NC + c
        w_base = wid * per_w
        bufs = ((buf_a0, buf_b0, buf_e0, sem_a0, sem_b0),
                (buf_a1, buf_b1, buf_e1, sem_a1, sem_b1))
        isems = (sem_i0, sem_i1)
        ssems = (sem_s0, sem_s1)

        def zrow(r, carry):
            for j in range(vecs_per_row):
                buf_e0[r, pl.ds(j * _LANES, _LANES)] = jnp.zeros(
                    (_LANES,), jnp.float32)
            return carry
        lax.fori_loop(0, _ECHK, zrow, 0)
        for i in range(max_rc_per_sub):
            chunk = i * _NS + s

            @pl.when(chunk < n_row_chunks)
            def _():
                pltpu.sync_copy(buf_e0,
                                m_sh.at[pl.ds(chunk * row_chk, row_chk)])
        plsc.subcore_barrier()

        def issue_idx(i, p):
            base = w_base + i * _ECHK
            pltpu.async_copy(src_hbm.at[pl.ds(base, _ECHK)],
                             idx_s2.at[p], isems[p])
            pltpu.async_copy(dst_hbm.at[pl.ds(base, _ECHK)],
                             idx_d2.at[p], isems[p])

        def wait_idx(p):
            pltpu.make_async_copy(src_hbm.at[pl.ds(0, _ECHK)],
                                  idx_s2.at[p], isems[p]).wait()
            pltpu.make_async_copy(dst_hbm.at[pl.ds(0, _ECHK)],
                                  idx_d2.at[p], isems[p]).wait()

        def issue_gather(p):
            ba, bb, _, sa, sb = bufs[p]
            pltpu.async_copy(a_hbm.at[idx_s2.at[p]], ba, sa)
            pltpu.async_copy(b_hbm.at[idx_d2.at[p]], bb, sb)

        def wait_gather(p):
            ba, bb, _, sa, sb = bufs[p]
            pltpu.make_async_copy(a_hbm.at[idx_s2.at[p]], ba, sa).wait()
            pltpu.make_async_copy(b_hbm.at[idx_d2.at[p]], bb, sb).wait()

        def compute(p):
            ba, bb, be, _, _ = bufs[p]
            for j in range(_ECHK // _LANES):
                sl = pl.ds(j * _LANES, _LANES)
                sidx2[p, sl] = idx_d2[p, sl]
            zero32 = jnp.zeros((2 * _LANES,), jnp.bfloat16)

            def row2(r2, rc):
                r0 = r2 * 2
                for u in range(2):
                    for j in range(d // (2 * _LANES)):
                        sl = pl.ds(j * _LANES, _LANES)
                        va = plsc.bitcast(ba[r0 + u, sl], jnp.bfloat16)
                        vb = plsc.bitcast(bb[r0 + u, sl], jnp.bfloat16)
                        ev = jnp.maximum(va + vb, zero32)
                        lo, hi = plsc.unpack(
                            ev, format=plsc.PackFormat.INTERLEAVED)
                        be[r0 + u, pl.ds(2 * j * _LANES, _LANES)] = lo
                        be[r0 + u, pl.ds((2 * j + 1) * _LANES, _LANES)] = hi
                return rc
            lax.fori_loop(0, _ECHK // 2, row2, 0)

        def scatter_start(p):
            be = bufs[p][2]
            pltpu.async_copy(be, m_sh.at[sidx2.at[p]], ssems[p], add=True)

        def scatter_wait(p):
            be = bufs[p][2]
            pltpu.make_async_copy(be, m_sh.at[sidx2.at[p]], ssems[p]).wait()

        issue_idx(0, 0)
        issue_idx(1, 1)
        wait_idx(0)
        issue_gather(0)

        def pair_body(k, carry):
            wait_gather(0)
            compute(0)
            scatter_start(0)
            issue_idx(2 * k + 2, 0)
            wait_idx(1)

            @pl.when(k > 0)
            def _():
                scatter_wait(1)
            issue_gather(1)
            wait_gather(1)
            compute(1)
            scatter_start(1)

            @pl.when(2 * k + 3 < n_chunks)
            def _():
                issue_idx(2 * k + 3, 1)
            wait_idx(0)
            scatter_wait(0)
            issue_gather(0)
            return carry
        lax.fori_loop(0, n_pairs, pair_body, 0)
        wait_gather(0)
        compute(0)
        scatter_start(0)
        scatter_wait(1)
        scatter_wait(0)
        plsc.subcore_barrier()

        for i in range(max_rc_per_sub):
            chunk = i * _NS + s

            @pl.when(chunk < n_row_chunks)
            def _():
                sl = pl.ds(chunk * row_chk, row_chk)
                pltpu.sync_copy(m_sh.at[sl], out_hbm.at[c, sl])

    return sc_edge


def kernel(x, edge_index, W_msg, b_msg, W_node, b_node):
    n, d = x.shape
    e = edge_index.shape[1]
    src = edge_index[0].astype(jnp.int32)
    dst = edge_index[1].astype(jnp.int32)
    perm = np.arange(d).reshape(d // 32, 2, 16).transpose(0, 2, 1).reshape(-1)
    w1 = W_msg[:d][:, perm]
    w2 = W_msg[d:][:, perm]
    wn1 = W_node[:d]
    wn2 = W_node[d:]
    bm = b_msg[perm].reshape(1, d)
    bn = b_node.reshape(1, d)
    blk = 1000 if n % 1000 == 0 else n

    sc_edge = _make_sc_edge(n, e, d)

    h = x
    for _ in range(NUM_STEPS):
        a, b = _msg_pre(h, w1, w2, bm, blk=blk)
        ai = jax.lax.bitcast_convert_type(a.reshape(n, d // 2, 2), jnp.int32)
        bi = jax.lax.bitcast_convert_type(b.reshape(n, d // 2, 2), jnp.int32)
        m2 = sc_edge(ai, bi, src, dst)
        h = _node_update(h, m2, wn1, wn2, bn, blk=blk)
    return h


# speedup vs baseline: 1.0109x; 1.0109x over previous
import functools

import jax
import jax.numpy as jnp
import numpy as np
from jax import lax
from jax.experimental import pallas as pl
from jax.experimental.pallas import tpu as pltpu
from jax.experimental.pallas import tpu_sc as plsc

NUM_STEPS = 3


def _msg_pre_body(h_ref, w1_ref, w2_ref, b_ref, a_ref, bb_ref):
    h = h_ref[...]
    a_ref[...] = jnp.dot(
        h, w1_ref[...], preferred_element_type=jnp.float32
    ).astype(jnp.bfloat16)
    bb_ref[...] = (
        jnp.dot(h, w2_ref[...], preferred_element_type=jnp.float32) + b_ref[...]
    ).astype(jnp.bfloat16)


def _node_body(h_ref, m_ref, w1_ref, w2_ref, b_ref, o_ref):
    m = m_ref[0] + m_ref[1]
    acc = jnp.dot(h_ref[...], w1_ref[...], preferred_element_type=jnp.float32)
    acc = acc + jnp.dot(m, w2_ref[...], preferred_element_type=jnp.float32)
    o_ref[...] = jnp.maximum(acc + b_ref[...], 0.0)


def _fused_body(h_ref, m_ref, wn1_ref, wn2_ref, bn_ref, w1_ref, w2_ref,
                bm_ref, o_ref, a_ref, bb_ref):
    m = m_ref[0] + m_ref[1]
    acc = jnp.dot(h_ref[...], wn1_ref[...], preferred_element_type=jnp.float32)
    acc = acc + jnp.dot(m, wn2_ref[...], preferred_element_type=jnp.float32)
    hn = jnp.maximum(acc + bn_ref[...], 0.0)
    o_ref[...] = hn
    a_ref[...] = jnp.dot(
        hn, w1_ref[...], preferred_element_type=jnp.float32
    ).astype(jnp.bfloat16)
    bb_ref[...] = (
        jnp.dot(hn, w2_ref[...], preferred_element_type=jnp.float32)
        + bm_ref[...]
    ).astype(jnp.bfloat16)


@functools.partial(jax.jit, static_argnames=("blk",))
def _fused_update_pre(h, m2, wn1, wn2, bn2d, w1, w2, bm2d, blk):
    n, d = h.shape
    grid = (n // blk,)
    wspec = pl.BlockSpec((d, d), lambda i: (0, 0))
    bspec = pl.BlockSpec((1, d), lambda i: (0, 0))
    rspec = pl.BlockSpec((blk, d), lambda i: (i, 0))
    return pl.pallas_call(
        _fused_body,
        grid=grid,
        in_specs=[
            rspec,
            pl.BlockSpec((2, blk, d), lambda i: (0, i, 0)),
            wspec, wspec, bspec, wspec, wspec, bspec,
        ],
        out_specs=[rspec, rspec, rspec],
        out_shape=[
            jax.ShapeDtypeStruct((n, d), jnp.float32),
            jax.ShapeDtypeStruct((n, d), jnp.bfloat16),
            jax.ShapeDtypeStruct((n, d), jnp.bfloat16),
        ],
    )(h, m2, wn1, wn2, bn2d, w1, w2, bm2d)


@functools.partial(jax.jit, static_argnames=("blk",))
def _msg_pre(h, w1, w2, b2d, blk):
    n, d = h.shape
    grid = (n // blk,)
    return pl.pallas_call(
        _msg_pre_body,
        grid=grid,
        in_specs=[
            pl.BlockSpec((blk, d), lambda i: (i, 0)),
            pl.BlockSpec((d, d), lambda i: (0, 0)),
            pl.BlockSpec((d, d), lambda i: (0, 0)),
            pl.BlockSpec((1, d), lambda i: (0, 0)),
        ],
        out_specs=[
            pl.BlockSpec((blk, d), lambda i: (i, 0)),
            pl.BlockSpec((blk, d), lambda i: (i, 0)),
        ],
        out_shape=[jax.ShapeDtypeStruct((n, d), jnp.bfloat16)] * 2,
    )(h, w1, w2, b2d)


@functools.partial(jax.jit, static_argnames=("blk",))
def _node_update(h, m2, w1, w2, b2d, blk):
    n, d = h.shape
    grid = (n // blk,)
    return pl.pallas_call(
        _node_body,
        grid=grid,
        in_specs=[
            pl.BlockSpec((blk, d), lambda i: (i, 0)),
            pl.BlockSpec((2, blk, d), lambda i: (0, i, 0)),
            pl.BlockSpec((d, d), lambda i: (0, 0)),
            pl.BlockSpec((d, d), lambda i: (0, 0)),
            pl.BlockSpec((1, d), lambda i: (0, 0)),
        ],
        out_specs=pl.BlockSpec((blk, d), lambda i: (i, 0)),
        out_shape=jax.ShapeDtypeStruct((n, d), jnp.float32),
    )(h, m2, w1, w2, b2d)


_NC = 2
_NS = 16
_LANES = 16
_ECHK = 80


def _make_sc_edge(n_nodes, n_edges, d):
    nw = _NC * _NS
    per_w = n_edges // nw
    n_chunks = per_w // _ECHK
    n_pairs = n_chunks // 2
    vecs_per_row = d // _LANES
    row_chk = _ECHK
    n_row_chunks = n_nodes // row_chk
    max_rc_per_sub = -(-n_row_chunks // _NS)

    mesh = plsc.VectorSubcoreMesh(core_axis_name="c", subcore_axis_name="s")

    @functools.partial(
        pl.kernel,
        mesh=mesh,
        compiler_params=pltpu.CompilerParams(
            needs_layout_passes=False, use_tc_tiling_on_sc=False),
        out_type=jax.ShapeDtypeStruct((_NC, n_nodes, d), jnp.float32),
        scratch_types=[
            pltpu.VMEM((2, _ECHK), jnp.int32),
            pltpu.VMEM((2, _ECHK), jnp.int32),
            pltpu.VMEM((_ECHK, d // 2), jnp.int32),
            pltpu.VMEM((_ECHK, d // 2), jnp.int32),
            pltpu.VMEM((_ECHK, d // 2), jnp.int32),
            pltpu.VMEM((_ECHK, d // 2), jnp.int32),
            pltpu.VMEM((_ECHK, d), jnp.float32),
            pltpu.VMEM((_ECHK, d), jnp.float32),
            pltpu.VMEM((2, _ECHK), jnp.int32),
            pltpu.VMEM_SHARED((n_nodes, d), jnp.float32),
            pltpu.SemaphoreType.DMA,
            pltpu.SemaphoreType.DMA,
            pltpu.SemaphoreType.DMA,
            pltpu.SemaphoreType.DMA,
            pltpu.SemaphoreType.DMA,
            pltpu.SemaphoreType.DMA,
            pltpu.SemaphoreType.DMA,
            pltpu.SemaphoreType.DMA,
        ],
    )
    def sc_edge(a_hbm, b_hbm, src_hbm, dst_hbm, out_hbm,
                idx_s2, idx_d2, buf_a0, buf_b0, buf_a1, buf_b1,
                buf_e0, buf_e1, sidx2, m_sh,
                sem_i0, sem_i1, sem_a0, sem_b0, sem_a1, sem_b1,
                sem_s0, sem_s1):
        c = lax.axis_index("c")
        s = lax.axis_index("s")
        wid = s * _NC + c
        w_base = wid * per_w
        bufs = ((buf_a0, buf_b0, buf_e0, sem_a0, sem_b0),
                (buf_a1, buf_b1, buf_e1, sem_a1, sem_b1))
        isems = (sem_i0, sem_i1)
        ssems = (sem_s0, sem_s1)

        def zrow(r, carry):
            for j in range(vecs_per_row):
                buf_e0[r, pl.ds(j * _LANES, _LANES)] = jnp.zeros(
                    (_LANES,), jnp.float32)
            return carry
        lax.fori_loop(0, _ECHK, zrow, 0)
        for i in range(max_rc_per_sub):
            chunk = i * _NS + s

            @pl.when(chunk < n_row_chunks)
            def _():
                pltpu.sync_copy(buf_e0,
                                m_sh.at[pl.ds(chunk * row_chk, row_chk)])
        plsc.subcore_barrier()

        def issue_idx(i, p):
            base = w_base + i * _ECHK
            pltpu.async_copy(src_hbm.at[pl.ds(base, _ECHK)],
                             idx_s2.at[p], isems[p])
            pltpu.async_copy(dst_hbm.at[pl.ds(base, _ECHK)],
                             idx_d2.at[p], isems[p])

        def wait_idx(p):
            pltpu.make_async_copy(src_hbm.at[pl.ds(0, _ECHK)],
                                  idx_s2.at[p], isems[p]).wait()
            pltpu.make_async_copy(dst_hbm.at[pl.ds(0, _ECHK)],
                                  idx_d2.at[p], isems[p]).wait()

        def issue_gather(p):
            ba, bb, _, sa, sb = bufs[p]
            pltpu.async_copy(a_hbm.at[idx_s2.at[p]], ba, sa)
            pltpu.async_copy(b_hbm.at[idx_d2.at[p]], bb, sb)

        def wait_gather(p):
            ba, bb, _, sa, sb = bufs[p]
            pltpu.make_async_copy(a_hbm.at[idx_s2.at[p]], ba, sa).wait()
            pltpu.make_async_copy(b_hbm.at[idx_d2.at[p]], bb, sb).wait()

        def compute(p):
            ba, bb, be, _, _ = bufs[p]
            for j in range(_ECHK // _LANES):
                sl = pl.ds(j * _LANES, _LANES)
                sidx2[p, sl] = idx_d2[p, sl]
            zero32 = jnp.zeros((2 * _LANES,), jnp.bfloat16)

            def row2(r2, rc):
                r0 = r2 * 2
                for u in range(2):
                    for j in range(d // (2 * _LANES)):
                        sl = pl.ds(j * _LANES, _LANES)
                        va = plsc.bitcast(ba[r0 + u, sl], jnp.bfloat16)
                        vb = plsc.bitcast(bb[r0 + u, sl], jnp.bfloat16)
                        ev = jnp.maximum(va + vb, zero32)
                        lo, hi = plsc.unpack(
                            ev, format=plsc.PackFormat.INTERLEAVED)
                        be[r0 + u, pl.ds(2 * j * _LANES, _LANES)] = lo
                        be[r0 + u, pl.ds((2 * j + 1) * _LANES, _LANES)] = hi
                return rc
            lax.fori_loop(0, _ECHK // 2, row2, 0)

        def scatter_start(p):
            be = bufs[p][2]
            pltpu.async_copy(be, m_sh.at[sidx2.at[p]], ssems[p], add=True)

        def scatter_wait(p):
            be = bufs[p][2]
            pltpu.make_async_copy(be, m_sh.at[sidx2.at[p]], ssems[p]).wait()

        issue_idx(0, 0)
        issue_idx(1, 1)
        wait_idx(0)
        issue_gather(0)

        def pair_body(k, carry):
            wait_gather(0)
            compute(0)
            scatter_start(0)
            issue_idx(2 * k + 2, 0)
            wait_idx(1)

            @pl.when(k > 0)
            def _():
                scatter_wait(1)
            issue_gather(1)
            wait_gather(1)
            compute(1)
            scatter_start(1)

            @pl.when(2 * k + 3 < n_chunks)
            def _():
                issue_idx(2 * k + 3, 1)
            wait_idx(0)
            scatter_wait(0)
            issue_gather(0)
            return carry
        lax.fori_loop(0, n_pairs, pair_body, 0)
        wait_gather(0)
        compute(0)
        scatter_start(0)
        scatter_wait(1)
        scatter_wait(0)
        plsc.subcore_barrier()

        for i in range(max_rc_per_sub):
            chunk = i * _NS + s

            @pl.when(chunk < n_row_chunks)
            def _():
                sl = pl.ds(chunk * row_chk, row_chk)
                pltpu.sync_copy(m_sh.at[sl], out_hbm.at[c, sl])

    return sc_edge


def kernel(x, edge_index, W_msg, b_msg, W_node, b_node):
    n, d = x.shape
    e = edge_index.shape[1]
    src = edge_index[0].astype(jnp.int32)
    dst = edge_index[1].astype(jnp.int32)
    perm = np.arange(d).reshape(d // 32, 2, 16).transpose(0, 2, 1).reshape(-1)
    w1 = W_msg[:d][:, perm]
    w2 = W_msg[d:][:, perm]
    wn1 = W_node[:d]
    wn2 = W_node[d:]
    bm = b_msg[perm].reshape(1, d)
    bn = b_node.reshape(1, d)
    blk = 1000 if n % 1000 == 0 else n

    sc_edge = _make_sc_edge(n, e, d)

    def _pack(a, b):
        ai = jax.lax.bitcast_convert_type(a.reshape(n, d // 2, 2), jnp.int32)
        bi = jax.lax.bitcast_convert_type(b.reshape(n, d // 2, 2), jnp.int32)
        return ai, bi

    h = x
    a, b = _msg_pre(h, w1, w2, bm, blk=blk)
    m2 = sc_edge(*_pack(a, b), src, dst)
    for _ in range(NUM_STEPS - 1):
        h, a, b = _fused_update_pre(h, m2, wn1, wn2, bn, w1, w2, bm, blk=blk)
        m2 = sc_edge(*_pack(a, b), src, dst)
    h = _node_update(h, m2, wn1, wn2, bn, blk=blk)
    return h


# speedup vs baseline: 1.6369x; 1.6193x over previous
import functools

import jax
import jax.numpy as jnp
import numpy as np
from jax import lax
from jax.experimental import pallas as pl
from jax.experimental.pallas import tpu as pltpu
from jax.experimental.pallas import tpu_sc as plsc

NUM_STEPS = 3


def _msg_pre_body(h_ref, w1_ref, w2_ref, b_ref, a_ref, bb_ref):
    h = h_ref[...]
    a_ref[...] = jnp.dot(h, w1_ref[...], preferred_element_type=jnp.float32)
    bb_ref[...] = (
        jnp.dot(h, w2_ref[...], preferred_element_type=jnp.float32) + b_ref[...]
    )


def _node_body(h_ref, m_ref, w1_ref, w2_ref, b_ref, o_ref):
    m = m_ref[0] + m_ref[1]
    acc = jnp.dot(h_ref[...], w1_ref[...], preferred_element_type=jnp.float32)
    acc = acc + jnp.dot(m, w2_ref[...], preferred_element_type=jnp.float32)
    o_ref[...] = jnp.maximum(acc + b_ref[...], 0.0)


def _fused_body(h_ref, m_ref, wn1_ref, wn2_ref, bn_ref, w1_ref, w2_ref,
                bm_ref, o_ref, a_ref, bb_ref):
    m = m_ref[0] + m_ref[1]
    acc = jnp.dot(h_ref[...], wn1_ref[...], preferred_element_type=jnp.float32)
    acc = acc + jnp.dot(m, wn2_ref[...], preferred_element_type=jnp.float32)
    hn = jnp.maximum(acc + bn_ref[...], 0.0)
    o_ref[...] = hn
    a_ref[...] = jnp.dot(hn, w1_ref[...], preferred_element_type=jnp.float32)
    bb_ref[...] = (
        jnp.dot(hn, w2_ref[...], preferred_element_type=jnp.float32)
        + bm_ref[...]
    )


@functools.partial(jax.jit, static_argnames=("blk",))
def _fused_update_pre(h, m2, wn1, wn2, bn2d, w1, w2, bm2d, blk):
    n, d = h.shape
    grid = (n // blk,)
    wspec = pl.BlockSpec((d, d), lambda i: (0, 0))
    bspec = pl.BlockSpec((1, d), lambda i: (0, 0))
    rspec = pl.BlockSpec((blk, d), lambda i: (i, 0))
    return pl.pallas_call(
        _fused_body,
        grid=grid,
        in_specs=[
            rspec,
            pl.BlockSpec((2, blk, d), lambda i: (0, i, 0)),
            wspec, wspec, bspec, wspec, wspec, bspec,
        ],
        out_specs=[rspec, rspec, rspec],
        out_shape=[
            jax.ShapeDtypeStruct((n, d), jnp.float32),
            jax.ShapeDtypeStruct((n, d), jnp.float32),
            jax.ShapeDtypeStruct((n, d), jnp.float32),
        ],
    )(h, m2, wn1, wn2, bn2d, w1, w2, bm2d)


@functools.partial(jax.jit, static_argnames=("blk",))
def _msg_pre(h, w1, w2, b2d, blk):
    n, d = h.shape
    grid = (n // blk,)
    return pl.pallas_call(
        _msg_pre_body,
        grid=grid,
        in_specs=[
            pl.BlockSpec((blk, d), lambda i: (i, 0)),
            pl.BlockSpec((d, d), lambda i: (0, 0)),
            pl.BlockSpec((d, d), lambda i: (0, 0)),
            pl.BlockSpec((1, d), lambda i: (0, 0)),
        ],
        out_specs=[
            pl.BlockSpec((blk, d), lambda i: (i, 0)),
            pl.BlockSpec((blk, d), lambda i: (i, 0)),
        ],
        out_shape=[jax.ShapeDtypeStruct((n, d), jnp.float32)] * 2,
    )(h, w1, w2, b2d)


@functools.partial(jax.jit, static_argnames=("blk",))
def _node_update(h, m2, w1, w2, b2d, blk):
    n, d = h.shape
    grid = (n // blk,)
    return pl.pallas_call(
        _node_body,
        grid=grid,
        in_specs=[
            pl.BlockSpec((blk, d), lambda i: (i, 0)),
            pl.BlockSpec((2, blk, d), lambda i: (0, i, 0)),
            pl.BlockSpec((d, d), lambda i: (0, 0)),
            pl.BlockSpec((d, d), lambda i: (0, 0)),
            pl.BlockSpec((1, d), lambda i: (0, 0)),
        ],
        out_specs=pl.BlockSpec((blk, d), lambda i: (i, 0)),
        out_shape=jax.ShapeDtypeStruct((n, d), jnp.float32),
    )(h, m2, w1, w2, b2d)


_NC = 2
_NS = 16
_LANES = 16
_ECHK = 80


def _make_sc_edge(n_nodes, n_edges, d):
    nw = _NC * _NS
    per_w = n_edges // nw
    n_chunks = per_w // _ECHK
    n_pairs = n_chunks // 2
    vecs_per_row = d // _LANES
    row_chk = _ECHK
    n_row_chunks = n_nodes // row_chk
    max_rc_per_sub = -(-n_row_chunks // _NS)

    mesh = plsc.VectorSubcoreMesh(core_axis_name="c", subcore_axis_name="s")

    @functools.partial(
        pl.kernel,
        mesh=mesh,
        out_type=jax.ShapeDtypeStruct((_NC, n_nodes, d), jnp.float32),
        scratch_types=[
            pltpu.VMEM((2, _ECHK), jnp.int32),
            pltpu.VMEM((2, _ECHK), jnp.int32),
            pltpu.VMEM((_ECHK, d), jnp.float32),
            pltpu.VMEM((_ECHK, d), jnp.float32),
            pltpu.VMEM((_ECHK, d), jnp.float32),
            pltpu.VMEM((_ECHK, d), jnp.float32),
            pltpu.VMEM((2, _ECHK), jnp.int32),
            pltpu.VMEM_SHARED((n_nodes, d), jnp.float32),
            pltpu.SemaphoreType.DMA,
            pltpu.SemaphoreType.DMA,
            pltpu.SemaphoreType.DMA,
            pltpu.SemaphoreType.DMA,
            pltpu.SemaphoreType.DMA,
            pltpu.SemaphoreType.DMA,
            pltpu.SemaphoreType.DMA,
            pltpu.SemaphoreType.DMA,
        ],
    )
    def sc_edge(a_hbm, b_hbm, src_hbm, dst_hbm, out_hbm,
                idx_s2, idx_d2, buf_a0, buf_b0, buf_a1, buf_b1, sidx2, m_sh,
                sem_i0, sem_i1, sem_a0, sem_b0, sem_a1, sem_b1,
                sem_s0, sem_s1):
        c = lax.axis_index("c")
        s = lax.axis_index("s")
        wid = s * _NC + c
        w_base = wid * per_w
        bufs = ((buf_a0, buf_b0, sem_a0, sem_b0),
                (buf_a1, buf_b1, sem_a1, sem_b1))
        isems = (sem_i0, sem_i1)
        ssems = (sem_s0, sem_s1)

        def zrow(r, carry):
            for j in range(vecs_per_row):
                buf_a0[r, pl.ds(j * _LANES, _LANES)] = jnp.zeros(
                    (_LANES,), jnp.float32)
            return carry
        lax.fori_loop(0, _ECHK, zrow, 0)
        for i in range(max_rc_per_sub):
            chunk = i * _NS + s

            @pl.when(chunk < n_row_chunks)
            def _():
                pltpu.sync_copy(buf_a0,
                                m_sh.at[pl.ds(chunk * row_chk, row_chk)])
        plsc.subcore_barrier()

        def issue_idx(i, p):
            base = w_base + i * _ECHK
            pltpu.async_copy(src_hbm.at[pl.ds(base, _ECHK)],
                             idx_s2.at[p], isems[p])
            pltpu.async_copy(dst_hbm.at[pl.ds(base, _ECHK)],
                             idx_d2.at[p], isems[p])

        def wait_idx(p):
            pltpu.make_async_copy(src_hbm.at[pl.ds(0, _ECHK)],
                                  idx_s2.at[p], isems[p]).wait()
            pltpu.make_async_copy(dst_hbm.at[pl.ds(0, _ECHK)],
                                  idx_d2.at[p], isems[p]).wait()

        def issue_gather(p):
            ba, bb, sa, sb = bufs[p]
            pltpu.async_copy(a_hbm.at[idx_s2.at[p]], ba, sa)
            pltpu.async_copy(b_hbm.at[idx_d2.at[p]], bb, sb)

        def wait_gather(p):
            ba, bb, sa, sb = bufs[p]
            pltpu.make_async_copy(a_hbm.at[idx_s2.at[p]], ba, sa).wait()
            pltpu.make_async_copy(b_hbm.at[idx_d2.at[p]], bb, sb).wait()

        def compute(p):
            ba, bb, _, _ = bufs[p]
            for j in range(_ECHK // _LANES):
                sl = pl.ds(j * _LANES, _LANES)
                sidx2[p, sl] = idx_d2[p, sl]

            def row4(r4, rc):
                r0 = r4 * 4
                for u in range(4):
                    for j in range(vecs_per_row):
                        sl = pl.ds(j * _LANES, _LANES)
                        ba[r0 + u, sl] = jnp.maximum(
                            ba[r0 + u, sl] + bb[r0 + u, sl], 0.0)
                return rc
            lax.fori_loop(0, _ECHK // 4, row4, 0)

        def scatter_start(p):
            ba = bufs[p][0]
            pltpu.async_copy(ba, m_sh.at[sidx2.at[p]], ssems[p], add=True)

        def scatter_wait(p):
            ba = bufs[p][0]
            pltpu.make_async_copy(ba, m_sh.at[sidx2.at[p]], ssems[p]).wait()

        issue_idx(0, 0)
        issue_idx(1, 1)
        wait_idx(0)
        issue_gather(0)

        def pair_body(k, carry):
            wait_gather(0)
            compute(0)
            scatter_start(0)
            issue_idx(2 * k + 2, 0)
            wait_idx(1)

            @pl.when(k > 0)
            def _():
                scatter_wait(1)
            issue_gather(1)
            wait_gather(1)
            compute(1)
            scatter_start(1)

            @pl.when(2 * k + 3 < n_chunks)
            def _():
                issue_idx(2 * k + 3, 1)
            wait_idx(0)
            scatter_wait(0)
            issue_gather(0)
            return carry
        lax.fori_loop(0, n_pairs, pair_body, 0)
        wait_gather(0)
        compute(0)
        scatter_start(0)
        scatter_wait(1)
        scatter_wait(0)
        plsc.subcore_barrier()

        for i in range(max_rc_per_sub):
            chunk = i * _NS + s

            @pl.when(chunk < n_row_chunks)
            def _():
                sl = pl.ds(chunk * row_chk, row_chk)
                pltpu.sync_copy(m_sh.at[sl], out_hbm.at[c, sl])

    return sc_edge


def kernel(x, edge_index, W_msg, b_msg, W_node, b_node):
    n, d = x.shape
    e = edge_index.shape[1]
    src = edge_index[0].astype(jnp.int32)
    dst = edge_index[1].astype(jnp.int32)
    w1 = W_msg[:d]
    w2 = W_msg[d:]
    wn1 = W_node[:d]
    wn2 = W_node[d:]
    bm = b_msg.reshape(1, d)
    bn = b_node.reshape(1, d)
    blk = 1000 if n % 1000 == 0 else n

    sc_edge = _make_sc_edge(n, e, d)

    h = x
    a, b = _msg_pre(h, w1, w2, bm, blk=blk)
    m2 = sc_edge(a, b, src, dst)
    for _ in range(NUM_STEPS - 1):
        h, a, b = _fused_update_pre(h, m2, wn1, wn2, bn, w1, w2, bm, blk=blk)
        m2 = sc_edge(a, b, src, dst)
    h = _node_update(h, m2, wn1, wn2, bn, blk=blk)
    return h
